# Initial kernel scaffold; baseline (speedup 1.0000x reference)
import jax
import jax.numpy as jnp
from jax.experimental import pallas as pl


def kernel(x, token_table, pos_table):
    raise NotImplementedError("write your pallas kernel here")


# speedup vs baseline: 4.2666x; 4.2666x over previous
import functools

import jax
import jax.numpy as jnp
from jax import lax
from jax.experimental import pallas as pl
from jax.experimental.pallas import tpu as pltpu
from jax.experimental.pallas import tpu_sc as plsc

VOCAB = 100000
SEQ = 200
EMB = 128
BATCH = 1024

NC = 2
NS = 16
NW = NC * NS
SEQ_PER_W = BATCH // NW
HALF = SEQ // 2

_mesh = plsc.VectorSubcoreMesh(
    core_axis_name="c", subcore_axis_name="s", num_cores=NC, num_subcores=NS
)


@functools.partial(
    pl.kernel,
    out_type=jax.ShapeDtypeStruct((BATCH * SEQ, EMB), jnp.float32),
    mesh=_mesh,
    scratch_types=[
        pltpu.VMEM((2 * SEQ_PER_W, HALF), jnp.int32),
        pltpu.VMEM((SEQ, EMB), jnp.float32),
        pltpu.VMEM((SEQ, EMB), jnp.float32),
        pltpu.SemaphoreType.DMA,
    ],
)
def _emb_kernel(idx_hbm, table_hbm, pos_hbm, out_hbm, idx_v, pos_v, buf, sem):
    wid = lax.axis_index("s") * NC + lax.axis_index("c")
    pltpu.sync_copy(idx_hbm.at[wid], idx_v)
    pltpu.sync_copy(pos_hbm, pos_v)

    @pl.loop(0, SEQ_PER_W)
    def _seq_loop(seq):
        g0 = pltpu.async_copy(
            table_hbm.at[idx_v.at[2 * seq]], buf.at[pl.ds(0, HALF)], sem
        )
        g1 = pltpu.async_copy(
            table_hbm.at[idx_v.at[2 * seq + 1]], buf.at[pl.ds(HALF, HALF)], sem
        )
        g0.wait()
        g1.wait()

        @pl.loop(0, SEQ)
        def _row_loop(r):
            for j in range(EMB // 16):
                sl = pl.ds(j * 16, 16)
                plsc.addupdate(buf.at[r, sl], pos_v[r, sl])

        base = wid * (SEQ_PER_W * SEQ) + seq * SEQ
        pltpu.sync_copy(buf, out_hbm.at[pl.ds(base, SEQ)])


def kernel(x, token_table, pos_table):
    idx = x.reshape(NW, 2 * SEQ_PER_W, HALF)
    out = _emb_kernel(idx, token_table, pos_table)
    return out.reshape(BATCH, SEQ, EMB)


# speedup vs baseline: 4.7036x; 1.1024x over previous
import functools

import jax
import jax.numpy as jnp
from jax import lax
from jax.experimental import pallas as pl
from jax.experimental.pallas import tpu as pltpu
from jax.experimental.pallas import tpu_sc as plsc

VOCAB = 100000
SEQ = 200
EMB = 128
BATCH = 1024

NC = 2
NS = 16
NW = NC * NS
SEQ_PER_W = BATCH // NW
HALF = SEQ // 2

_mesh = plsc.VectorSubcoreMesh(
    core_axis_name="c", subcore_axis_name="s", num_cores=NC, num_subcores=NS
)


@functools.partial(
    pl.kernel,
    out_type=jax.ShapeDtypeStruct((BATCH * SEQ, EMB), jnp.float32),
    mesh=_mesh,
    scratch_types=[
        pltpu.VMEM((2 * SEQ_PER_W, HALF), jnp.int32),
        pltpu.VMEM((SEQ, EMB), jnp.float32),
        pltpu.VMEM((2, SEQ, EMB), jnp.float32),
        pltpu.SemaphoreType.DMA,
        pltpu.SemaphoreType.DMA,
    ],
)
def _emb_kernel(
    idx_hbm, table_hbm, pos_hbm, out_hbm, idx_v, pos_v, buf, gsem0, gsem1
):
    gsem = (gsem0, gsem1)
    wid = lax.axis_index("s") * NC + lax.axis_index("c")
    pltpu.sync_copy(idx_hbm.at[wid], idx_v)
    pltpu.sync_copy(pos_hbm, pos_v)

    def start_gather(seq, b):
        g0 = pltpu.async_copy(
            table_hbm.at[idx_v.at[2 * seq]], buf.at[b, pl.ds(0, HALF)], gsem[b]
        )
        g1 = pltpu.async_copy(
            table_hbm.at[idx_v.at[2 * seq + 1]], buf.at[b, pl.ds(HALF, HALF)], gsem[b]
        )
        return g0, g1

    @pl.loop(0, SEQ_PER_W // 2)
    def _pair_loop(i):
        gathers = [start_gather(2 * i + b, b) for b in range(2)]
        for b in range(2):
            seq = 2 * i + b
            g0, g1 = gathers[b]
            g0.wait()
            g1.wait()

            @pl.loop(0, SEQ)
            def _row_loop(r):
                for j in range(EMB // 16):
                    sl = pl.ds(j * 16, 16)
                    plsc.addupdate(buf.at[b, r, sl], pos_v[r, sl])

            base = wid * (SEQ_PER_W * SEQ) + seq * SEQ
            pltpu.sync_copy(buf.at[b], out_hbm.at[pl.ds(base, SEQ)])


def kernel(x, token_table, pos_table):
    idx = x.reshape(NW, 2 * SEQ_PER_W, HALF)
    out = _emb_kernel(idx, token_table, pos_table)
    return out.reshape(BATCH, SEQ, EMB)
